# baseline (speedup 1.0000x reference)
import functools

import jax
import jax.numpy as jnp
from jax import lax
from jax.experimental import pallas as pl
from jax.experimental.pallas import tpu as pltpu
from jax.experimental.pallas import tpu_sc as plsc

B, T, C = 32, 32768, 64
L = 16
G = C // L
TT = 1024
TT1 = TT + 1
NT = T // TT
NCH = G * NT
SUBV = 64
NSUB = TT // SUBV
NEG = float("-inf")


def _tree_max(vs):
    while len(vs) > 1:
        nxt = [jnp.maximum(vs[2 * i], vs[2 * i + 1]) for i in range(len(vs) // 2)]
        if len(vs) % 2:
            nxt.append(vs[-1])
        vs = nxt
    return vs[0]


def _build():
    mesh = plsc.VectorSubcoreMesh(core_axis_name="c", subcore_axis_name="s")
    nc = mesh.num_cores

    @functools.partial(
        pl.kernel,
        out_type=jax.ShapeDtypeStruct((B, C), jnp.float32),
        mesh=mesh,
        compiler_params=pltpu.CompilerParams(needs_layout_passes=False),
        scratch_types=[
            pltpu.VMEM((2, L, TT1), jnp.float32),
            pltpu.VMEM((5, L), jnp.float32),
            pltpu.VMEM((C,), jnp.float32),
            pltpu.SemaphoreType.DMA,
            pltpu.SemaphoreType.DMA,
        ],
    )
    def k(xt_hbm, out_hbm, buf, state, outbuf, sem0, sem1):
        b = lax.axis_index("s") * nc + lax.axis_index("c")
        sems = (sem0, sem1)
        riota = lax.iota(jnp.int32, L)

        def start(ci, which):
            cg = lax.shift_right_logical(ci, 5)
            t0 = (ci & 31) * TT
            pltpu.async_copy(
                xt_hbm.at[b, pl.ds(cg * L, L), pl.ds(t0, TT)],
                buf.at[which, :, pl.ds(0, TT)], sems[which])

        def wait(which):
            pltpu.make_async_copy(
                xt_hbm.at[b, pl.ds(0, L), pl.ds(0, TT)],
                buf.at[which, :, pl.ds(0, TT)], sems[which]).wait()

        def reset_state():
            for i in range(5):
                state[i] = jnp.full((L,), NEG, jnp.float32)

        def process(which):
            whichv = jnp.full((L,), which, jnp.int32)

            def gather(colv, kk):
                return plsc.load_gather(buf, [whichv, riota, colv + kk])

            def sub_body(s, carry):
                col0 = s * SUBV
                colv = jnp.full((L,), col0, jnp.int32)
                thr = state[4]
                masks = []
                for j in range(4):
                    m16 = _tree_max(
                        [gather(colv, j * 16 + kk) for kk in range(16)])
                    masks.append(m16 > thr)
                any4 = (masks[0] | masks[1]) | (masks[2] | masks[3])
                cnt = plsc.all_reduce_population_count(any4)

                @pl.when(cnt[0] > 0)
                def _():
                    for j in range(4):
                        cj = plsc.all_reduce_population_count(masks[j])

                        @pl.when(cj[0] > 0)
                        def _():
                            m = [state[i] for i in range(5)]
                            for kk in range(16):
                                v = gather(colv, j * 16 + kk)
                                for i in range(5):
                                    hi = jnp.maximum(m[i], v)
                                    v = jnp.minimum(m[i], v)
                                    m[i] = hi
                            for i in range(5):
                                state[i] = m[i]
                return carry

            lax.fori_loop(0, NSUB, sub_body, 0)

        def finalize(ci):
            cg = lax.shift_right_logical(ci, 5)
            acc = state[0]
            for i in range(1, 5):
                acc = acc + state[i]
            outbuf[pl.ds(cg * L, L)] = acc * jnp.float32(0.2)

        def handle(which, ci):
            @pl.when((ci & 31) == 0)
            def _():
                reset_state()

            wait(which)
            process(which)

            @pl.when((ci & 31) == 31)
            def _():
                finalize(ci)

        start(0, 0)

        def pair(p, carry):
            ci0 = 2 * p
            start(ci0 + 1, 1)
            handle(0, ci0)

            @pl.when(ci0 + 2 < NCH)
            def _():
                start(ci0 + 2, 0)

            handle(1, ci0 + 1)
            return carry

        lax.fori_loop(0, NCH // 2, pair, 0)

        pltpu.sync_copy(outbuf, out_hbm.at[b])

    return k


def kernel(inputs):
    xt = jnp.transpose(inputs, (0, 2, 1))
    return _build()(xt)
